# Initial kernel scaffold; baseline (speedup 1.0000x reference)
import jax
import jax.numpy as jnp
from jax.experimental import pallas as pl


def kernel(x, table):
    raise NotImplementedError("write your pallas kernel here")


# speedup vs baseline: 4.2204x; 4.2204x over previous
import functools
import numpy as np
import jax
import jax.numpy as jnp
from jax import lax
from jax.experimental import pallas as pl
from jax.experimental.pallas import tpu as pltpu
from jax.experimental.pallas import tpu_sc as plsc

_VOCAB = 100000
_DIM = 64
_BATCH = 4096
_SEQ = 200

_NC = 2
_NS = 16
_L = 16
_NW = _NC * _NS
_CH = 128
_PER_W = (_BATCH * _SEQ) // _NW
_NCHUNK = _PER_W // _CH
_NBUF = 8
_G = 4
_NGROUP = _NCHUNK // _NBUF


def _positional_encoding_np(seq, d_model):
    pos = np.arange(seq, dtype=np.float32)[:, None]
    i = np.arange(0, d_model, 2, dtype=np.float32)
    div = np.power(10000.0, i / d_model)
    pe = np.zeros((seq, d_model), dtype=np.float32)
    pe[:, 0::2] = np.sin(pos / div)
    pe[:, 1::2] = np.cos(pos / div)
    return pe


_PE_EXT = np.concatenate([_positional_encoding_np(_SEQ, _DIM)] * 2, axis=0)


@functools.partial(
    pl.kernel,
    out_type=jax.ShapeDtypeStruct((_BATCH * _SEQ, _DIM), jnp.float32),
    mesh=plsc.VectorSubcoreMesh(core_axis_name="c", subcore_axis_name="s"),
    scratch_types=[
        pltpu.VMEM((_NCHUNK, _CH), jnp.int32),
        pltpu.VMEM((2 * _SEQ, _DIM), jnp.float32),
        pltpu.VMEM((_NBUF, _CH, _DIM), jnp.float32),
    ]
    + [pltpu.SemaphoreType.DMA] * (2 * _NBUF),
    compiler_params=pltpu.CompilerParams(use_tc_tiling_on_sc=False),
)
def _embed_kernel(x_hbm, pe_hbm, table_hbm, out_hbm, idx_v, pe_v, rows_v, *sems):
    gsems = sems[:_NBUF]
    ssems = sems[_NBUF:]
    wid = lax.axis_index("s") * _NC + lax.axis_index("c")
    base = wid * _PER_W
    pltpu.sync_copy(x_hbm.at[wid], idx_v)
    pltpu.sync_copy(pe_hbm, pe_v)

    def gather_start(c, b):
        pltpu.async_copy(table_hbm.at[idx_v.at[c]], rows_v.at[b], gsems[b])

    def gather_wait(c, b):
        pltpu.make_async_copy(
            table_hbm.at[idx_v.at[c]], rows_v.at[b], gsems[b]
        ).wait()

    def store_start(c, b):
        pltpu.async_copy(
            rows_v.at[b], out_hbm.at[pl.ds(base + c * _CH, _CH)], ssems[b]
        )

    def store_wait(b):
        pltpu.make_async_copy(
            rows_v.at[b], out_hbm.at[pl.ds(base, _CH)], ssems[b]
        ).wait()

    def add_pe(c, b):
        p0 = lax.rem(c * _CH, _SEQ)

        @plsc.parallel_loop(0, _CH, 1, unroll=8)
        def _(i):
            for j in range(_DIM // _L):
                plsc.addupdate(
                    rows_v.at[b, i, pl.ds(j * _L, _L)],
                    pe_v[p0 + i, pl.ds(j * _L, _L)],
                )

    def do_chunk(c, b, issue_next, wait_store):
        gather_wait(c, b)
        add_pe(c, b)
        store_start(c, b)
        if issue_next:
            b2 = (b + _G) % _NBUF
            if wait_store:
                store_wait(b2)
            gather_start(c + _G, b2)

    for b in range(_G):
        gather_start(b, b)

    for b in range(_NBUF):
        do_chunk(b, b, issue_next=True, wait_store=(b >= _G))

    def group_body(g, carry):
        c0 = g * _NBUF
        for b in range(_NBUF):
            do_chunk(c0 + b, b, issue_next=True, wait_store=True)
        return carry

    lax.fori_loop(1, _NGROUP - 1, group_body, 0)

    c0 = (_NGROUP - 1) * _NBUF
    for b in range(_NBUF):
        do_chunk(c0 + b, b, issue_next=(b < _NBUF - _G), wait_store=True)

    for b in range(_NBUF - _G, _NBUF):
        store_wait(b)


def kernel(x, table):
    xf = x.reshape(_NW, _NCHUNK, _CH)
    out = _embed_kernel(xf, _PE_EXT, table)
    return out.reshape(_BATCH, _SEQ, _DIM)
